# Initial kernel scaffold; baseline (speedup 1.0000x reference)
import jax
import jax.numpy as jnp
from jax.experimental import pallas as pl


def kernel(x, edge_index, W0, b0, g0, be0, W1, b1, g1, be1, W2, b2, g2, be2, W3, b3, g3, be3, l1W, l1b, l2W, l2b):
    raise NotImplementedError("write your pallas kernel here")


# speedup vs baseline: 7.3244x; 7.3244x over previous
import functools

import jax
import jax.numpy as jnp
from jax import lax
from jax.experimental import pallas as pl
from jax.experimental.pallas import tpu as pltpu
from jax.experimental.pallas import tpu_sc as plsc

N = 10000
E = 320000
IN = 128
H = 256
OUT = 128

NC = 2
NS = 16
CH = 128
EPAD = 323584
NROWS = 10112

_mesh = plsc.VectorSubcoreMesh(core_axis_name="c", subcore_axis_name="s")


@functools.partial(
    pl.kernel,
    out_type=jax.ShapeDtypeStruct((NC, NROWS, 16), jnp.float32),
    mesh=_mesh,
    scratch_types=[
        pltpu.VMEM_SHARED((NROWS, 16), jnp.float32),
        pltpu.VMEM((CH,), jnp.int32),
        pltpu.VMEM((CH, 16), jnp.float32),
    ],
)
def _deg(dstp, zeros_init, ones_rows, degp, acc, idx_v, ones_v):
    c = lax.axis_index("c")
    s = lax.axis_index("s")
    w = c * NS + s
    per_w = EPAD // (NC * NS)
    nchunks = per_w // CH

    pltpu.sync_copy(ones_rows, ones_v)

    @pl.when(s == 0)
    def _():
        pltpu.sync_copy(zeros_init, acc)

    plsc.subcore_barrier()

    def body(j, carry):
        base = w * per_w + j * CH
        pltpu.sync_copy(dstp.at[pl.ds(base, CH)], idx_v)
        pltpu.sync_copy(ones_v, acc.at[idx_v], add=True)
        return carry

    lax.fori_loop(0, nchunks, body, 0)
    plsc.subcore_barrier()

    @pl.when(s == 0)
    def _():
        pltpu.sync_copy(acc, degp.at[c])


def _make_agg(wh):
    per_tile = EPAD // NS
    nchunks = per_tile // CH
    rpt = NROWS // NS

    @functools.partial(
        pl.kernel,
        out_type=(
            jax.ShapeDtypeStruct((NROWS, wh), jnp.float32),
            jax.ShapeDtypeStruct((NROWS, wh), jnp.float32),
        ),
        mesh=_mesh,
        scratch_types=[
            pltpu.VMEM_SHARED((NROWS, wh), jnp.float32),
            pltpu.VMEM((CH,), jnp.int32),
            pltpu.VMEM((CH,), jnp.int32),
            pltpu.VMEM((CH, wh), jnp.float32),
            pltpu.SemaphoreType.DMA,
        ],
    )
    def agg(uL, uR, srcp, dstp, aggL, aggR, acc, sidx, didx, rows, gsem):
        c = lax.axis_index("c")
        s = lax.axis_index("s")

        def half(u_hbm, out_hbm):
            @pl.when(s == 0)
            def _():
                pltpu.sync_copy(u_hbm, acc.at[pl.ds(0, N)])

            plsc.subcore_barrier()

            def body(j, carry):
                base = s * per_tile + j * CH
                pltpu.sync_copy(srcp.at[pl.ds(base, CH)], sidx)
                pltpu.sync_copy(dstp.at[pl.ds(base, CH)], didx)
                pltpu.async_copy(u_hbm.at[sidx], rows, gsem).wait()
                pltpu.sync_copy(rows, acc.at[didx], add=True)
                return carry

            lax.fori_loop(0, nchunks, body, 0)
            plsc.subcore_barrier()
            pltpu.sync_copy(
                acc.at[pl.ds(s * rpt, rpt)], out_hbm.at[pl.ds(s * rpt, rpt)]
            )

        @pl.when(c == 0)
        def _():
            half(uL, aggL)

        @pl.when(c == 1)
        def _():
            half(uR, aggR)

    return agg


_agg128 = _make_agg(H // 2)


@functools.partial(
    pl.kernel,
    out_type=(
        jax.ShapeDtypeStruct((NROWS, IN), jnp.float32),
        jax.ShapeDtypeStruct((NROWS, IN), jnp.float32),
    ),
    mesh=_mesh,
    scratch_types=[
        pltpu.VMEM_SHARED((NROWS, IN), jnp.float32),
        pltpu.VMEM((CH,), jnp.int32),
        pltpu.VMEM((CH,), jnp.int32),
        pltpu.VMEM((CH, IN), jnp.float32),
        pltpu.SemaphoreType.DMA,
    ],
)
def _agg_l0(u, zinit, srcp, dstp, out0, out1, acc, sidx, didx, rows, gsem):
    c = lax.axis_index("c")
    s = lax.axis_index("s")
    half_edges = EPAD // NC
    per_tile = half_edges // NS
    nchunks = per_tile // CH
    rpt = NROWS // NS

    @pl.when(s == 0)
    def _():
        @pl.when(c == 0)
        def _():
            pltpu.sync_copy(u, acc.at[pl.ds(0, N)])

        @pl.when(c == 1)
        def _():
            pltpu.sync_copy(zinit, acc)

    plsc.subcore_barrier()

    def body(j, carry):
        base = c * half_edges + s * per_tile + j * CH
        pltpu.sync_copy(srcp.at[pl.ds(base, CH)], sidx)
        pltpu.sync_copy(dstp.at[pl.ds(base, CH)], didx)
        pltpu.async_copy(u.at[sidx], rows, gsem).wait()
        pltpu.sync_copy(rows, acc.at[didx], add=True)
        return carry

    lax.fori_loop(0, nchunks, body, 0)
    plsc.subcore_barrier()

    @pl.when(c == 0)
    def _():
        pltpu.sync_copy(acc.at[pl.ds(s * rpt, rpt)], out0.at[pl.ds(s * rpt, rpt)])

    @pl.when(c == 1)
    def _():
        pltpu.sync_copy(acc.at[pl.ds(s * rpt, rpt)], out1.at[pl.ds(s * rpt, rpt)])


def _prep_body(deg_ref, x_ref, dis_ref, u_ref):
    deg = deg_ref[:, 0:1] + deg_ref[:, 1:2] + 1.0
    dis = lax.rsqrt(deg)
    dis_ref[...] = dis
    u_ref[...] = x_ref[...] * dis


def _prep(degt, x):
    return pl.pallas_call(
        _prep_body,
        out_shape=(
            jax.ShapeDtypeStruct((N, 1), jnp.float32),
            jax.ShapeDtypeStruct((N, IN), jnp.float32),
        ),
    )(degt, x)


BLK = 1000
NBLK = N // BLK


def _make_mm_body(combine_sum):
    def _mm_body(aggL_ref, aggR_ref, dis_ref, w_ref, b_ref, z_ref, stats_ref, acc_ref):
        i = pl.program_id(0)
        if combine_sum:
            t = (aggL_ref[...] + aggR_ref[...]) * dis_ref[...]
        else:
            t = jnp.concatenate([aggL_ref[...], aggR_ref[...]], axis=1) * dis_ref[...]
        z = jnp.dot(t, w_ref[...], preferred_element_type=jnp.float32) + b_ref[...]
        z_ref[...] = z

        @pl.when(i == 0)
        def _():
            acc_ref[...] = jnp.zeros_like(acc_ref)

        acc_ref[0:1, :] += jnp.sum(z, axis=0, keepdims=True)
        acc_ref[1:2, :] += jnp.sum(z * z, axis=0, keepdims=True)
        stats_ref[...] = acc_ref[...]

    return _mm_body


def _matmul_stats(aggL, aggR, dis, w, b, combine_sum=False):
    wh = aggL.shape[1]
    win = w.shape[0]
    return pl.pallas_call(
        _make_mm_body(combine_sum),
        grid=(NBLK,),
        in_specs=[
            pl.BlockSpec((BLK, wh), lambda i: (i, 0)),
            pl.BlockSpec((BLK, wh), lambda i: (i, 0)),
            pl.BlockSpec((BLK, 1), lambda i: (i, 0)),
            pl.BlockSpec((win, H), lambda i: (0, 0)),
            pl.BlockSpec((1, H), lambda i: (0, 0)),
        ],
        out_specs=(
            pl.BlockSpec((BLK, H), lambda i: (i, 0)),
            pl.BlockSpec((2, H), lambda i: (0, 0)),
        ),
        out_shape=(
            jax.ShapeDtypeStruct((N, H), jnp.float32),
            jax.ShapeDtypeStruct((2, H), jnp.float32),
        ),
        scratch_shapes=[pltpu.VMEM((2, H), jnp.float32)],
    )(aggL, aggR, dis, w, b)


def _bn_body(z_ref, stats_ref, g_ref, be_ref, dis_ref, m_ref, mo_ref, uL_ref, uR_ref):
    mu = stats_ref[0:1, :] * (1.0 / N)
    var = stats_ref[1:2, :] * (1.0 / N) - mu * mu
    inv = lax.rsqrt(var + 1e-5)
    h = jnp.maximum((z_ref[...] - mu) * (inv * g_ref[...]) + be_ref[...], 0.0)
    mo_ref[...] = jnp.maximum(m_ref[...], h)
    u = h * dis_ref[...]
    uL_ref[...] = u[:, : H // 2]
    uR_ref[...] = u[:, H // 2 :]


def _bn_relu_max(z, stats, g, be, dis, m):
    return pl.pallas_call(
        _bn_body,
        grid=(NBLK,),
        in_specs=[
            pl.BlockSpec((BLK, H), lambda i: (i, 0)),
            pl.BlockSpec((2, H), lambda i: (0, 0)),
            pl.BlockSpec((1, H), lambda i: (0, 0)),
            pl.BlockSpec((1, H), lambda i: (0, 0)),
            pl.BlockSpec((BLK, 1), lambda i: (i, 0)),
            pl.BlockSpec((BLK, H), lambda i: (i, 0)),
        ],
        out_specs=(
            pl.BlockSpec((BLK, H), lambda i: (i, 0)),
            pl.BlockSpec((BLK, H // 2), lambda i: (i, 0)),
            pl.BlockSpec((BLK, H // 2), lambda i: (i, 0)),
        ),
        out_shape=(
            jax.ShapeDtypeStruct((N, H), jnp.float32),
            jax.ShapeDtypeStruct((N, H // 2), jnp.float32),
            jax.ShapeDtypeStruct((N, H // 2), jnp.float32),
        ),
    )(z, stats, g, be, dis, m)


def _head_body(m_ref, w1_ref, b1_ref, w2_ref, b2_ref, o_ref):
    h = jnp.maximum(
        jnp.dot(m_ref[...], w1_ref[...], preferred_element_type=jnp.float32)
        + b1_ref[...],
        0.0,
    )
    o_ref[...] = (
        jnp.dot(h, w2_ref[...], preferred_element_type=jnp.float32) + b2_ref[...]
    )


def _head(m, w1, b1, w2, b2):
    return pl.pallas_call(
        _head_body,
        grid=(NBLK,),
        in_specs=[
            pl.BlockSpec((BLK, H), lambda i: (i, 0)),
            pl.BlockSpec((H, H), lambda i: (0, 0)),
            pl.BlockSpec((1, H), lambda i: (0, 0)),
            pl.BlockSpec((H, OUT), lambda i: (0, 0)),
            pl.BlockSpec((1, OUT), lambda i: (0, 0)),
        ],
        out_specs=pl.BlockSpec((BLK, OUT), lambda i: (i, 0)),
        out_shape=jax.ShapeDtypeStruct((N, OUT), jnp.float32),
    )(m, w1, b1, w2, b2)


def kernel(x, edge_index, W0, b0, g0, be0, W1, b1, g1, be1, W2, b2, g2, be2,
           W3, b3, g3, be3, l1W, l1b, l2W, l2b):
    src = edge_index[0].astype(jnp.int32)
    dst = edge_index[1].astype(jnp.int32)
    pad = EPAD - E
    srcp = jnp.concatenate([src, jnp.zeros((pad,), jnp.int32)])
    dstp = jnp.concatenate([dst, jnp.full((pad,), N, jnp.int32)])

    zeros_init = jnp.zeros((NROWS, 16), jnp.float32)
    ones_rows = jnp.ones((CH, 16), jnp.float32)
    degp = _deg(dstp, zeros_init, ones_rows)
    degt = degp[:, :N, 0].T

    dis, u0 = _prep(degt, x)
    zinit = jnp.zeros((NROWS, IN), jnp.float32)

    params = [(W0, b0, g0, be0), (W1, b1, g1, be1),
              (W2, b2, g2, be2), (W3, b3, g3, be3)]
    m = jnp.zeros((N, H), jnp.float32)
    uL = uR = None
    for i, (Wl, bl, gl, bel) in enumerate(params):
        if i == 0:
            aggL, aggR = _agg_l0(u0, zinit, srcp, dstp)
        else:
            aggL, aggR = _agg128(uL, uR, srcp, dstp)
        aggL, aggR = aggL[:N], aggR[:N]
        z, stats = _matmul_stats(
            aggL, aggR, dis, Wl, bl.reshape(1, H), combine_sum=(i == 0)
        )
        m, uL, uR = _bn_relu_max(
            z, stats, gl.reshape(1, H), bel.reshape(1, H), dis, m
        )

    return _head(m, l1W, l1b.reshape(1, H), l2W, l2b.reshape(1, OUT))
